# baseline (speedup 1.0000x reference)
import jax
import jax.numpy as jnp
from jax import lax
from jax.experimental import pallas as pl

_B = 1024
_IOU_T = 0.5
_SCORE_T = 0.05
_MAXDET = 100
_C = 256


def _iou_gt(x1a, y1a, x2a, y2a, aa, x1b, y1b, x2b, y2b, ab):
    ltx = jnp.maximum(x1a, x1b)
    lty = jnp.maximum(y1a, y1b)
    rbx = jnp.minimum(x2a, x2b)
    rby = jnp.minimum(y2a, y2b)
    w = jnp.maximum(rbx - ltx, 0.0)
    h = jnp.maximum(rby - lty, 0.0)
    inter = w * h
    union = aa + ab - inter
    iou = inter / jnp.maximum(union, 1e-9)
    return iou > _IOU_T


def _nms_step(rows_ref, cols_ref, cblk_ref, ss_ref, vals_ref, out_ref,
              keep_ref):
    np_ = keep_ref.shape[1]
    nb = np_ // _B
    bi = pl.program_id(0)
    f32 = jnp.float32
    bf16 = jnp.bfloat16

    @pl.when(bi == 0)
    def _():
        keep_ref[...] = (ss_ref[...] > _SCORE_T).astype(f32)

    rb = rows_ref[...]
    rx1, ry1, rx2, ry2 = rb[:, 0:1], rb[:, 1:2], rb[:, 2:3], rb[:, 3:4]
    ra = (rx2 - rx1) * (ry2 - ry1)

    bb = cblk_ref[...]
    bx1, by1, bx2, by2 = bb[0:1, :], bb[1:2, :], bb[2:3, :], bb[3:4, :]
    ba = (bx2 - bx1) * (by2 - by1)

    m_sub = _iou_gt(rx1, ry1, rx2, ry2, ra, bx1, by1, bx2, by2, ba)
    ii = lax.broadcasted_iota(jnp.int32, (_B, _B), 0)
    jj = lax.broadcasted_iota(jnp.int32, (_B, _B), 1)
    m_sub = (m_sub & (jj > ii)).astype(bf16)

    k_init = keep_ref[0:1, pl.ds(bi * _B, _B)]

    def cond(c):
        _, changed, t = c
        return jnp.logical_and(changed, t < _B + 2)

    def body(c):
        k, _, t = c
        supp = lax.dot_general(
            k.astype(bf16), m_sub, (((1,), (0,)), ((), ())),
            preferred_element_type=f32)
        k_new = jnp.where(supp > 0.5, 0.0, k_init)
        return k_new, jnp.any(k_new != k), t + 1

    k_fin, _, _ = lax.while_loop(
        cond, body, (k_init, jnp.bool_(True), jnp.int32(0)))
    keep_ref[0:1, pl.ds(bi * _B, _B)] = k_fin

    cnt = jnp.sum(k_fin)
    nrb = _B // 128
    kf2 = k_fin.reshape(nrb, 128).astype(bf16)
    pi = lax.broadcasted_iota(jnp.int32, (128, 128), 0)
    pj = lax.broadcasted_iota(jnp.int32, (128, 128), 1)
    u128 = (pi <= pj).astype(bf16)
    within = lax.dot_general(
        kf2, u128, (((1,), (0,)), ((), ())), preferred_element_type=f32)
    rowtot = within[:, 127:128]
    ti = lax.broadcasted_iota(jnp.int32, (nrb, nrb), 0)
    tj = lax.broadcasted_iota(jnp.int32, (nrb, nrb), 1)
    tlow = (tj < ti).astype(bf16)
    carry = lax.dot_general(
        tlow, rowtot.astype(bf16), (((1,), (0,)), ((), ())),
        preferred_element_type=f32)
    rank_inc = (within + carry).reshape(1, _B)
    rank0 = jnp.where(k_fin > 0.5, rank_inc - 1.0, -1.0)

    rows_f = rows_ref[...]
    cc = lax.broadcasted_iota(jnp.int32, (_C, _B), 0).astype(f32)

    def chunk(c):
        t, _ = c
        oh = (cc == rank0 - t * float(_C)).astype(f32)
        rows_c = lax.dot_general(
            oh, rows_f, (((1,), (0,)), ((), ())),
            preferred_element_type=f32,
            precision=lax.Precision.HIGHEST)
        gx1, gy1 = rows_c[:, 0:1], rows_c[:, 1:2]
        gx2, gy2 = rows_c[:, 2:3], rows_c[:, 3:4]
        ga = (gx2 - gx1) * (gy2 - gy1)
        ones_c = jnp.ones((1, _C), bf16)

        def tail(bj, carry2):
            cb = cols_ref[:, pl.ds(bj * _B, _B)]
            cx1, cy1 = cb[0:1, :], cb[1:2, :]
            cx2, cy2 = cb[2:3, :], cb[3:4, :]
            ca = (cx2 - cx1) * (cy2 - cy1)
            m = _iou_gt(gx1, gy1, gx2, gy2, ga, cx1, cy1, cx2, cy2, ca)
            supp = lax.dot_general(
                ones_c, m.astype(bf16), (((1,), (0,)), ((), ())),
                preferred_element_type=f32)
            kv = keep_ref[0:1, pl.ds(bj * _B, _B)]
            keep_ref[0:1, pl.ds(bj * _B, _B)] = jnp.where(
                supp > 0.5, 0.0, kv)
            return carry2

        lax.fori_loop(bi + 1, nb, tail, 0)
        return t + 1, 0

    lax.while_loop(lambda c: c[0] * _C < cnt, chunk, (jnp.float32(0.0), 0))

    @pl.when(bi == nb - 1)
    def _():
        keep = keep_ref[...]
        nr = np_ // 128
        k2 = keep.reshape(nr, 128).astype(bf16)
        pi = lax.broadcasted_iota(jnp.int32, (128, 128), 0)
        pj = lax.broadcasted_iota(jnp.int32, (128, 128), 1)
        u128 = (pi <= pj).astype(bf16)
        within = lax.dot_general(
            k2, u128, (((1,), (0,)), ((), ())),
            preferred_element_type=f32)
        rowtot = within[:, 127:128]
        ti = lax.broadcasted_iota(jnp.int32, (nr, nr), 0)
        tj = lax.broadcasted_iota(jnp.int32, (nr, nr), 1)
        tlow = (tj < ti).astype(bf16)
        carry = lax.dot_general(
            tlow, rowtot.astype(bf16), (((1,), (0,)), ((), ())),
            preferred_element_type=f32)
        kept_cnt = (within + carry).reshape(1, np_)
        total_kept = kept_cnt[0:1, np_ - 1:np_]
        col = lax.broadcasted_iota(jnp.int32, (1, np_), 1).astype(f32)
        not_cnt = (col + 1.0) - kept_cnt
        slot = jnp.where(keep > 0.5, kept_cnt - 1.0, total_kept + not_cnt - 1.0)
        rr = lax.broadcasted_iota(jnp.int32, (_MAXDET, np_), 0).astype(f32)
        onehot = (rr == slot).astype(f32)
        out_ref[...] = lax.dot_general(
            onehot, vals_ref[...], (((1,), (0,)), ((), ())),
            preferred_element_type=f32,
            precision=lax.Precision.HIGHEST)


def kernel(boxes, scores):
    n = boxes.shape[0]
    np_ = ((n + _B - 1) // _B) * _B
    nb = np_ // _B

    s = jnp.where(scores > _SCORE_T, scores, -1.0)
    neg_s, x1, y1, x2, y2 = lax.sort(
        (-s, boxes[:, 0], boxes[:, 1], boxes[:, 2], boxes[:, 3]),
        num_keys=1, is_stable=True)
    ss = -neg_s
    pad = jnp.full((np_ - n,), 0.0, jnp.float32)
    x1, y1, x2, y2 = (jnp.concatenate([c, pad]) for c in (x1, y1, x2, y2))
    ss_pad = jnp.concatenate([ss, jnp.full((np_ - n,), -1.0, jnp.float32)])
    bs_t = jnp.stack([x1, y1, x2, y2], axis=0)
    bs_pad = bs_t.T
    vals = jnp.concatenate(
        [bs_pad, ss_pad[:, None], jnp.zeros((np_, 3), jnp.float32)], axis=1)

    out, _ = pl.pallas_call(
        _nms_step,
        grid=(nb,),
        in_specs=[
            pl.BlockSpec((_B, 4), lambda i: (i, 0)),
            pl.BlockSpec((4, np_), lambda i: (0, 0)),
            pl.BlockSpec((4, _B), lambda i: (0, i)),
            pl.BlockSpec((1, np_), lambda i: (0, 0)),
            pl.BlockSpec((np_, 8), lambda i: (0, 0)),
        ],
        out_specs=[
            pl.BlockSpec((_MAXDET, 8), lambda i: (0, 0)),
            pl.BlockSpec((1, np_), lambda i: (0, 0)),
        ],
        out_shape=[
            jax.ShapeDtypeStruct((_MAXDET, 8), jnp.float32),
            jax.ShapeDtypeStruct((1, np_), jnp.float32),
        ],
    )(bs_pad, bs_t, bs_t, ss_pad[None, :], vals)

    return out[:, :5]


# speedup vs baseline: 3.4728x; 3.4728x over previous
import jax
import jax.numpy as jnp
from jax import lax
from jax.experimental import pallas as pl
from jax.experimental.pallas import tpu as pltpu

_B = 1024
_IOU_T = 0.5
_SCORE_T = 0.05
_MAXDET = 100


def _iou_gt(x1a, y1a, x2a, y2a, aa, x1b, y1b, x2b, y2b, ab):
    ltx = jnp.maximum(x1a, x1b)
    lty = jnp.maximum(y1a, y1b)
    rbx = jnp.minimum(x2a, x2b)
    rby = jnp.minimum(y2a, y2b)
    w = jnp.maximum(rbx - ltx, 0.0)
    h = jnp.maximum(rby - lty, 0.0)
    inter = w * h
    union = aa + ab - inter
    iou = inter / jnp.maximum(union, 1e-9)
    return iou > _IOU_T


def _nms_step(rows_ref, cols_ref, cblk_ref, ss_ref, vals_ref, out_ref,
              keep_ref, cnt_ref):
    np_ = keep_ref.shape[1]
    nb = np_ // _B
    bi = pl.program_id(0)
    f32 = jnp.float32
    bf16 = jnp.bfloat16

    @pl.when(bi == 0)
    def _():
        keep_ref[...] = (ss_ref[...] > _SCORE_T).astype(f32)
        cnt_ref[0] = 0.0

    @pl.when(cnt_ref[0] < float(_MAXDET))
    def _():
        rb = rows_ref[...]
        rx1, ry1, rx2, ry2 = rb[:, 0:1], rb[:, 1:2], rb[:, 2:3], rb[:, 3:4]
        ra = (rx2 - rx1) * (ry2 - ry1)

        bb = cblk_ref[...]
        bx1, by1, bx2, by2 = bb[0:1, :], bb[1:2, :], bb[2:3, :], bb[3:4, :]
        ba = (bx2 - bx1) * (by2 - by1)

        m_sub = _iou_gt(rx1, ry1, rx2, ry2, ra, bx1, by1, bx2, by2, ba)
        ii = lax.broadcasted_iota(jnp.int32, (_B, _B), 0)
        jj = lax.broadcasted_iota(jnp.int32, (_B, _B), 1)
        m_sub = (m_sub & (jj > ii)).astype(bf16)

        k_init = keep_ref[0:1, pl.ds(bi * _B, _B)]

        def cond(c):
            _, changed, t = c
            return jnp.logical_and(changed, t < _B + 2)

        def body(c):
            k, _, t = c
            supp = lax.dot_general(
                k.astype(bf16), m_sub, (((1,), (0,)), ((), ())),
                preferred_element_type=f32)
            k_new = jnp.where(supp > 0.5, 0.0, k_init)
            return k_new, jnp.any(k_new != k), t + 1

        k_fin, _, _ = lax.while_loop(
            cond, body, (k_init, jnp.bool_(True), jnp.int32(0)))
        keep_ref[0:1, pl.ds(bi * _B, _B)] = k_fin
        cnt_ref[0] = cnt_ref[0] + jnp.sum(k_fin)
        k_fin_bf = k_fin.astype(bf16)

        def tail(bj, carry):
            cb = cols_ref[:, pl.ds(bj * _B, _B)]
            cx1, cy1 = cb[0:1, :], cb[1:2, :]
            cx2, cy2 = cb[2:3, :], cb[3:4, :]
            ca = (cx2 - cx1) * (cy2 - cy1)
            m = _iou_gt(rx1, ry1, rx2, ry2, ra, cx1, cy1, cx2, cy2, ca)
            supp = lax.dot_general(
                k_fin_bf, m.astype(bf16), (((1,), (0,)), ((), ())),
                preferred_element_type=f32)
            kv = keep_ref[0:1, pl.ds(bj * _B, _B)]
            keep_ref[0:1, pl.ds(bj * _B, _B)] = jnp.where(
                supp > 0.5, 0.0, kv)
            return carry

        @pl.when(cnt_ref[0] < float(_MAXDET))
        def _():
            lax.fori_loop(bi + 1, nb, tail, 0)

    @pl.when(bi == nb - 1)
    def _():
        keep = keep_ref[...]
        nr = np_ // 128
        k2 = keep.reshape(nr, 128).astype(bf16)
        pi = lax.broadcasted_iota(jnp.int32, (128, 128), 0)
        pj = lax.broadcasted_iota(jnp.int32, (128, 128), 1)
        u128 = (pi <= pj).astype(bf16)
        within = lax.dot_general(
            k2, u128, (((1,), (0,)), ((), ())),
            preferred_element_type=f32)
        rowtot = within[:, 127:128]
        ti = lax.broadcasted_iota(jnp.int32, (nr, nr), 0)
        tj = lax.broadcasted_iota(jnp.int32, (nr, nr), 1)
        tlow = (tj < ti).astype(bf16)
        carry = lax.dot_general(
            tlow, rowtot.astype(bf16), (((1,), (0,)), ((), ())),
            preferred_element_type=f32)
        kept_cnt = (within + carry).reshape(1, np_)
        total_kept = kept_cnt[0:1, np_ - 1:np_]
        col = lax.broadcasted_iota(jnp.int32, (1, np_), 1).astype(f32)
        not_cnt = (col + 1.0) - kept_cnt
        slot = jnp.where(keep > 0.5, kept_cnt - 1.0,
                         total_kept + not_cnt - 1.0)
        rr = lax.broadcasted_iota(jnp.int32, (_MAXDET, np_), 0).astype(f32)
        onehot = (rr == slot).astype(f32)
        g = lax.dot_general(
            onehot, vals_ref[...], (((1,), (0,)), ((), ())),
            preferred_element_type=f32,
            precision=lax.Precision.HIGHEST)
        kept_scores = jnp.where(keep > 0.5, ss_ref[...], -1.0)
        sc = lax.dot_general(
            onehot, kept_scores, (((1,), (1,)), ((), ())),
            preferred_element_type=f32,
            precision=lax.Precision.HIGHEST)
        cm = lax.broadcasted_iota(jnp.int32, (1, 8), 1)
        out_ref[...] = g + sc * (cm == 4).astype(f32)


def kernel(boxes, scores):
    n = boxes.shape[0]
    np_ = ((n + _B - 1) // _B) * _B
    nb = np_ // _B

    s = jnp.where(scores > _SCORE_T, scores, -1.0)
    neg_s, x1, y1, x2, y2 = lax.sort(
        (-s, boxes[:, 0], boxes[:, 1], boxes[:, 2], boxes[:, 3]),
        num_keys=1, is_stable=True)
    ss = -neg_s
    pad = jnp.full((np_ - n,), 0.0, jnp.float32)
    x1, y1, x2, y2 = (jnp.concatenate([c, pad]) for c in (x1, y1, x2, y2))
    ss_pad = jnp.concatenate([ss, jnp.full((np_ - n,), -1.0, jnp.float32)])
    bs_t = jnp.stack([x1, y1, x2, y2], axis=0)
    bs_pad = bs_t.T
    vals = jnp.concatenate(
        [bs_pad, jnp.zeros((np_, 4), jnp.float32)], axis=1)

    out, _ = pl.pallas_call(
        _nms_step,
        grid=(nb,),
        in_specs=[
            pl.BlockSpec((_B, 4), lambda i: (i, 0)),
            pl.BlockSpec((4, np_), lambda i: (0, 0)),
            pl.BlockSpec((4, _B), lambda i: (0, i)),
            pl.BlockSpec((1, np_), lambda i: (0, 0)),
            pl.BlockSpec((np_, 8), lambda i: (0, 0)),
        ],
        out_specs=[
            pl.BlockSpec((_MAXDET, 8), lambda i: (0, 0)),
            pl.BlockSpec((1, np_), lambda i: (0, 0)),
        ],
        out_shape=[
            jax.ShapeDtypeStruct((_MAXDET, 8), jnp.float32),
            jax.ShapeDtypeStruct((1, np_), jnp.float32),
        ],
        scratch_shapes=[pltpu.SMEM((1,), jnp.float32)],
    )(bs_pad, bs_t, bs_t, ss_pad[None, :], vals)

    return out[:, :5]
